# baseline (speedup 1.0000x reference)
import functools

import jax
import jax.numpy as jnp
from jax import lax
from jax.experimental import pallas as pl
from jax.experimental.pallas import tpu as pltpu
from jax.experimental.pallas import tpu_sc as plsc

NC = 2
NS = 16
L = 16
NW = NC * NS
BATCH = 16384
EMBED = 16
BPW = BATCH // NW


def _sc_gather_dot(uidx, cidx, uemb, ubias, cemb, cbias):
  mesh = plsc.VectorSubcoreMesh(core_axis_name="c", subcore_axis_name="s")

  @functools.partial(
      pl.kernel,
      out_type=(
          jax.ShapeDtypeStruct((NW, L), jnp.float32),
          jax.ShapeDtypeStruct((BATCH,), jnp.float32),
      ),
      mesh=mesh,
      compiler_params=pltpu.CompilerParams(use_tc_tiling_on_sc=False),
      scratch_types=(
          pltpu.VMEM((BPW,), jnp.int32),
          pltpu.VMEM((BPW, L), jnp.float32),
          pltpu.VMEM((BPW, L), jnp.float32),
          pltpu.VMEM((BPW,), jnp.float32),
          pltpu.VMEM((BPW,), jnp.float32),
          pltpu.VMEM((L,), jnp.float32),
          pltpu.VMEM((BPW,), jnp.float32),
          pltpu.SemaphoreType.DMA,
      ),
  )
  def k(uidx_hbm, cidx_hbm, uemb_hbm, ubias_hbm, cemb_hbm, cbias_hbm,
        part_hbm, bsum_hbm,
        idx_v, urows_v, crows_v, ub_v, cb_v, acc_v, bsum_v, sem):
    wid = lax.axis_index("s") * NC + lax.axis_index("c")
    base = wid * BPW

    pltpu.sync_copy(uidx_hbm.at[pl.ds(base, BPW)], idx_v)
    u_rows_cp = pltpu.async_copy(uemb_hbm.at[idx_v], urows_v, sem)
    u_bias_cp = pltpu.async_copy(ubias_hbm.at[idx_v], ub_v, sem)
    u_rows_cp.wait()
    u_bias_cp.wait()

    pltpu.sync_copy(cidx_hbm.at[pl.ds(base, BPW)], idx_v)
    c_rows_cp = pltpu.async_copy(cemb_hbm.at[idx_v], crows_v, sem)
    c_bias_cp = pltpu.async_copy(cbias_hbm.at[idx_v], cb_v, sem)
    c_rows_cp.wait()
    c_bias_cp.wait()

    def dot_body(i, acc):
      return acc + urows_v[i, :] * crows_v[i, :]

    acc = lax.fori_loop(0, BPW, dot_body, jnp.zeros((L,), jnp.float32))
    acc_v[...] = acc
    pltpu.sync_copy(acc_v, part_hbm.at[wid])

    def bias_body(i, carry):
      bsum_v[pl.ds(i * L, L)] = ub_v[pl.ds(i * L, L)] + cb_v[pl.ds(i * L, L)]
      return carry

    lax.fori_loop(0, BPW // L, bias_body, 0)
    pltpu.sync_copy(bsum_v, bsum_hbm.at[pl.ds(base, BPW)])

  return k(uidx, cidx, uemb, ubias, cemb, cbias)


def _tc_finalize(partials, bsum2d):
  def body(p_ref, b_ref, o_ref):
    s = jnp.sum(p_ref[...])
    o_ref[...] = jax.nn.sigmoid(s + b_ref[...])

  return pl.pallas_call(
      body,
      out_shape=jax.ShapeDtypeStruct(bsum2d.shape, jnp.float32),
  )(partials, bsum2d)


def kernel(inputs, user_embedding, user_bias, cellphone_embedding,
           cellphone_bias):
  uidx = inputs[:, 0].astype(jnp.int32)
  cidx = inputs[:, 1].astype(jnp.int32)
  ub = user_bias.reshape(-1)
  cb = cellphone_bias.reshape(-1)

  partials, bsum = _sc_gather_dot(
      uidx, cidx, user_embedding, ub, cellphone_embedding, cb)
  out = _tc_finalize(partials, bsum.reshape(128, 128))
  return out.reshape(BATCH, 1)


# speedup vs baseline: 1.3747x; 1.3747x over previous
import functools

import jax
import jax.numpy as jnp
from jax import lax
from jax.experimental import pallas as pl
from jax.experimental.pallas import tpu as pltpu
from jax.experimental.pallas import tpu_sc as plsc

NC = 2
NS = 16
L = 16
NW = NC * NS
BATCH = 16384
EMBED = 16
BPW = BATCH // NW


def _sc_gather_dot(uidx, cidx, uemb, ubias, cemb, cbias, dummy):
  mesh = plsc.VectorSubcoreMesh(core_axis_name="c", subcore_axis_name="s")

  @functools.partial(
      pl.kernel,
      out_type=(
          jax.ShapeDtypeStruct((NW, L), jnp.float32),
          jax.ShapeDtypeStruct((BATCH,), jnp.float32),
      ),
      mesh=mesh,
      scratch_types=(
          pltpu.VMEM((BPW,), jnp.int32),
          pltpu.VMEM((BPW,), jnp.int32),
          pltpu.VMEM((BPW // 8, 128), jnp.float32),
          pltpu.VMEM((BPW // 8, 128), jnp.float32),
          pltpu.VMEM((BPW,), jnp.float32),
          pltpu.VMEM((BPW,), jnp.float32),
          pltpu.VMEM((L,), jnp.float32),
          pltpu.VMEM((BPW,), jnp.float32),
          pltpu.SemaphoreType.DMA,
          pltpu.SemaphoreType.DMA,
          pltpu.SemaphoreType.DMA,
      ),
  )
  def k(uidx_hbm, cidx_hbm, uemb_hbm, ubias_hbm, cemb_hbm, cbias_hbm,
        dummy_hbm, part_hbm, bsum_hbm,
        uidx_v, cidx_v, urows_v, crows_v, ub_v, cb_v, acc_v, bsum_v,
        sem_u, sem_c, sem_b):
    wid = lax.axis_index("s") * NC + lax.axis_index("c")
    base = wid * BPW

    pltpu.sync_copy(uidx_hbm.at[pl.ds(base, BPW)], uidx_v)
    pltpu.sync_copy(cidx_hbm.at[pl.ds(base, BPW)], cidx_v)

    ub_cp = pltpu.async_copy(ubias_hbm.at[uidx_v], ub_v, sem_b)
    cb_cp = pltpu.async_copy(cbias_hbm.at[cidx_v], cb_v, sem_b)

    def fire(ci, carry):
      uvec = uidx_v[pl.ds(ci * L, L)]
      cvec = cidx_v[pl.ds(ci * L, L)]
      for l in range(L):
        o = ci * L + l
        row = o // 8
        col = (o % 8) * EMBED
        pltpu.async_copy(
            uemb_hbm.at[uvec[l]], urows_v.at[row, pl.ds(col, EMBED)], sem_u)
        pltpu.async_copy(
            cemb_hbm.at[cvec[l]], crows_v.at[row, pl.ds(col, EMBED)], sem_c)
      return carry

    lax.fori_loop(0, BPW // L, fire, 0)

    pltpu.make_async_copy(dummy_hbm, urows_v, sem_u).wait()
    pltpu.make_async_copy(dummy_hbm, crows_v, sem_c).wait()

    def dot_body(i, acc):
      return (acc + urows_v[i // 8, pl.ds((i % 8) * EMBED, L)]
              * crows_v[i // 8, pl.ds((i % 8) * EMBED, L)])

    acc = lax.fori_loop(0, BPW, dot_body, jnp.zeros((L,), jnp.float32))
    acc_v[...] = acc
    pltpu.sync_copy(acc_v, part_hbm.at[wid])

    ub_cp.wait()
    cb_cp.wait()

    def bias_body(i, carry):
      bsum_v[pl.ds(i * L, L)] = ub_v[pl.ds(i * L, L)] + cb_v[pl.ds(i * L, L)]
      return carry

    lax.fori_loop(0, BPW // L, bias_body, 0)
    pltpu.sync_copy(bsum_v, bsum_hbm.at[pl.ds(base, BPW)])

  return k(uidx, cidx, uemb, ubias, cemb, cbias, dummy)


def _tc_finalize(partials, bsum2d):
  def body(p_ref, b_ref, o_ref):
    s = jnp.sum(p_ref[...])
    o_ref[...] = jax.nn.sigmoid(s + b_ref[...])

  return pl.pallas_call(
      body,
      out_shape=jax.ShapeDtypeStruct(bsum2d.shape, jnp.float32),
  )(partials, bsum2d)


def kernel(inputs, user_embedding, user_bias, cellphone_embedding,
           cellphone_bias):
  uidx = inputs[:, 0].astype(jnp.int32)
  cidx = inputs[:, 1].astype(jnp.int32)
  ub = user_bias.reshape(-1)
  cb = cellphone_bias.reshape(-1)

  dummy = jnp.zeros((BPW // 8, 128), jnp.float32)
  partials, bsum = _sc_gather_dot(
      uidx, cidx, user_embedding, ub, cellphone_embedding, cb, dummy)
  out = _tc_finalize(partials, bsum.reshape(128, 128))
  return out.reshape(BATCH, 1)
